# baseline (speedup 1.0000x reference)
import functools

import jax
import jax.numpy as jnp
from jax import lax
from jax.experimental import pallas as pl
from jax.experimental.pallas import tpu as pltpu
from jax.experimental.pallas import tpu_sc as plsc

N0 = 100000
N1 = 16384
N2 = 4096
D = 256

NC = 2
NS = 16
NW = NC * NS
L = 16


def _make_segmean(E, N_dst, R=128, B=128):
    num_ranges = N_dst // R
    ranges_per_tile = num_ranges // NW
    assert num_ranges % NW == 0 and E % B == 0 and B % L == 0
    DCH = D // L
    G = B // L

    mesh = plsc.VectorSubcoreMesh(
        core_axis_name="c", subcore_axis_name="s", num_cores=NC, num_subcores=NS
    )

    @functools.partial(
        pl.kernel,
        mesh=mesh,
        out_type=jax.ShapeDtypeStruct((N_dst, D), jnp.float32),
        scratch_types=[
            pltpu.VMEM((B,), jnp.int32),
            pltpu.VMEM((B,), jnp.int32),
            pltpu.VMEM((B, D), jnp.float32),
            pltpu.VMEM((R, D), jnp.float32),
            pltpu.VMEM((R,), jnp.float32),
            pltpu.VMEM((2 * num_ranges + L,), jnp.int32),
            pltpu.SemaphoreType.DMA,
        ],
    )
    def seg(table, src, dst, bounds, inv, out, idx_v, dst_v, rows, acc, inv_v, bounds_v, sem):
        cidx = lax.axis_index("c")
        sidx = lax.axis_index("s")
        wid = sidx * NC + cidx
        pltpu.sync_copy(bounds, bounds_v)
        zeros16 = jnp.zeros((L,), jnp.float32)

        for k in range(ranges_per_tile):
            ridx = wid + NW * k
            base = ridx * R

            def zero_row(r, _):
                for j in range(DCH):
                    acc[r, pl.ds(L * j, L)] = zeros16
                return 0

            lax.fori_loop(0, R, zero_row, 0)

            bv = bounds_v[pl.ds(2 * ridx, L)]
            c0 = bv[0]
            c1 = bv[1]

            def chunk_body(ci, _):
                off = ci * B
                pltpu.sync_copy(src.at[pl.ds(off, B)], idx_v)
                pltpu.sync_copy(dst.at[pl.ds(off, B)], dst_v)
                pltpu.async_copy(table.at[idx_v], rows, sem).wait()

                def group_body(g, _):
                    dv = dst_v[pl.ds(L * g, L)]
                    for l in range(L):
                        d = dv[l]
                        r = d - base

                        @pl.when((d >= base) & (d < base + R))
                        def _():
                            i = L * g + l
                            for j in range(DCH):
                                plsc.addupdate(
                                    acc.at[r, pl.ds(L * j, L)],
                                    rows[i, pl.ds(L * j, L)],
                                )

                    return 0

                lax.fori_loop(0, G, group_body, 0)
                return 0

            lax.fori_loop(c0, c1, chunk_body, 0)

            pltpu.sync_copy(inv.at[pl.ds(base, R)], inv_v)

            def mean_body(m, _):
                iv = inv_v[pl.ds(L * m, L)]
                for l in range(L):
                    s = iv[l]
                    r = L * m + l
                    for j in range(DCH):
                        acc[r, pl.ds(L * j, L)] = acc[r, pl.ds(L * j, L)] * s
                return 0

            lax.fori_loop(0, R // L, mean_body, 0)
            pltpu.sync_copy(acc, out.at[pl.ds(base, R)])

    return seg


def _segment_meta(dst, N_dst, R, B):
    num_ranges = N_dst // R
    grid = jnp.arange(0, N_dst + 1, R, dtype=jnp.int32)
    edges = jnp.searchsorted(dst, grid).astype(jnp.int32)
    c0 = edges[:-1] // B
    c1 = (edges[1:] + B - 1) // B
    bounds = jnp.stack([c0, c1], axis=1).reshape(-1)
    bounds = jnp.concatenate([bounds, jnp.zeros((L,), jnp.int32)])
    starts = jnp.searchsorted(dst, jnp.arange(N_dst + 1, dtype=jnp.int32)).astype(
        jnp.int32
    )
    cnt = (starts[1:] - starts[:-1]).astype(jnp.float32)
    inv = 1.0 / jnp.maximum(cnt, 1.0)
    return bounds, inv


def _make_sage_matmul(M, relu, MB=1024):
    def body(x_ref, n_ref, ws_ref, wn_ref, b_ref, o_ref):
        acc = jnp.dot(x_ref[...], ws_ref[...], preferred_element_type=jnp.float32)
        acc = acc + jnp.dot(n_ref[...], wn_ref[...], preferred_element_type=jnp.float32)
        acc = acc + b_ref[...]
        if relu:
            acc = jnp.maximum(acc, 0.0)
        o_ref[...] = acc

    return pl.pallas_call(
        body,
        grid=(M // MB,),
        in_specs=[
            pl.BlockSpec((MB, D), lambda i: (i, 0)),
            pl.BlockSpec((MB, D), lambda i: (i, 0)),
            pl.BlockSpec((D, D), lambda i: (0, 0)),
            pl.BlockSpec((D, D), lambda i: (0, 0)),
            pl.BlockSpec((1, D), lambda i: (0, 0)),
        ],
        out_specs=pl.BlockSpec((MB, D), lambda i: (i, 0)),
        out_shape=jax.ShapeDtypeStruct((M, D), jnp.float32),
    )


def kernel(inputs, src1, dst1, src2, dst2, W_self1, W_neigh1, b1, W_self2, W_neigh2, b2):
    src1 = src1.astype(jnp.int32)
    dst1 = dst1.astype(jnp.int32)
    src2 = src2.astype(jnp.int32)
    dst2 = dst2.astype(jnp.int32)

    R = 128
    B = 128
    seg1 = _make_segmean(src1.shape[0], N1, R, B)
    seg2 = _make_segmean(src2.shape[0], N2, R, B)

    bounds1, inv1 = _segment_meta(dst1, N1, R, B)
    hn1 = seg1(inputs, src1, dst1, bounds1, inv1)
    h1 = _make_sage_matmul(N1, relu=True)(
        inputs[:N1], hn1, W_self1, W_neigh1, b1.reshape(1, D)
    )

    bounds2, inv2 = _segment_meta(dst2, N2, R, B)
    hn2 = seg2(h1, src2, dst2, bounds2, inv2)
    out = _make_sage_matmul(N2, relu=False)(
        h1[:N2], hn2, W_self2, W_neigh2, b2.reshape(1, D)
    )
    return out


# speedup vs baseline: 14.1047x; 14.1047x over previous
import functools

import jax
import jax.numpy as jnp
from jax import lax
from jax.experimental import pallas as pl
from jax.experimental.pallas import tpu as pltpu
from jax.experimental.pallas import tpu_sc as plsc

N0 = 100000
N1 = 16384
N2 = 4096
D = 256

NC = 2
NS = 16
NW = NC * NS
L = 16


def _make_segmean(E, N_dst, R=128, B=128):
    num_ranges = N_dst // R
    ranges_per_tile = num_ranges // NW
    nchunks = E // B
    assert num_ranges % NW == 0 and E % B == 0 and nchunks % L == 0
    DCH = D // L
    G = B // L

    mesh = plsc.VectorSubcoreMesh(
        core_axis_name="c", subcore_axis_name="s", num_cores=NC, num_subcores=NS
    )

    @functools.partial(
        pl.kernel,
        mesh=mesh,
        out_type=jax.ShapeDtypeStruct((N_dst, D), jnp.float32),
        scratch_types=[
            pltpu.VMEM((2, B), jnp.int32),
            pltpu.VMEM((2, B), jnp.int32),
            pltpu.VMEM((2, B, D), jnp.float32),
            pltpu.VMEM((R, D), jnp.float32),
            pltpu.VMEM((R,), jnp.float32),
            pltpu.VMEM((nchunks,), jnp.int32),
            pltpu.SemaphoreType.DMA,
            pltpu.SemaphoreType.DMA,
            pltpu.SemaphoreType.DMA,
            pltpu.SemaphoreType.DMA,
            pltpu.SemaphoreType.DMA,
            pltpu.SemaphoreType.DMA,
        ],
    )
    def seg(
        table, src, dst, samp, out,
        idx_v, dst_v, rows, acc, cnt, samp_v,
        gsem0, gsem1, isem0, isem1, dsem0, dsem1,
    ):
        gsems = [gsem0, gsem1]
        isems = [isem0, isem1]
        dsems = [dsem0, dsem1]
        cidx = lax.axis_index("c")
        sidx = lax.axis_index("s")
        wid = sidx * NC + cidx
        pltpu.sync_copy(samp, samp_v)
        zeros16 = jnp.zeros((L,), jnp.float32)
        iota16 = lax.broadcasted_iota(jnp.int32, (L,), 0)

        def rank(x):
            def body(m, tot_v):
                v = samp_v[pl.ds(L * m, L)]
                ones = jnp.where(v < x, jnp.int32(1), jnp.int32(0))
                return tot_v + ones

            tot_v = lax.fori_loop(0, nchunks // L, body, jnp.zeros((L,), jnp.int32))
            tot = tot_v[0]
            for l in range(1, L):
                tot = tot + tot_v[l]
            return tot

        def stage_idx(ci, b):
            pltpu.async_copy(src.at[pl.ds(ci * B, B)], idx_v.at[b], isems[b])

        def stage_dst(ci, b):
            pltpu.async_copy(dst.at[pl.ds(ci * B, B)], dst_v.at[b], dsems[b])

        def wait_stage(ci, b):
            pltpu.make_async_copy(src.at[pl.ds(ci * B, B)], idx_v.at[b], isems[b]).wait()
            pltpu.make_async_copy(dst.at[pl.ds(ci * B, B)], dst_v.at[b], dsems[b]).wait()

        def fire_gather(b):
            pltpu.async_copy(table.at[idx_v.at[b]], rows.at[b], gsems[b])

        def wait_gather(b):
            pltpu.make_async_copy(table.at[idx_v.at[b]], rows.at[b], gsems[b]).wait()

        def accumulate(b, base):
            def group_body(g, _):
                dv = dst_v[b, pl.ds(L * g, L)]
                for l in range(L):
                    d = dv[l]
                    r = d - base

                    @pl.when((d >= base) & (d < base + R))
                    def _():
                        lane = lax.bitwise_and(r, L - 1)
                        rhi = r - lane
                        onehot = jnp.where(iota16 == lane, 1.0, 0.0)
                        plsc.addupdate(cnt.at[pl.ds(rhi, L)], onehot)
                        i = L * g + l
                        for j in range(DCH):
                            plsc.addupdate(
                                acc.at[r, pl.ds(L * j, L)],
                                rows[b, i, pl.ds(L * j, L)],
                            )

                return 0

            lax.fori_loop(0, G, group_body, 0)

        def range_body(k, _unused):
            ridx = wid + NW * k
            base = ridx * R

            def zero_row(r, _):
                for j in range(DCH):
                    acc[r, pl.ds(L * j, L)] = zeros16
                return 0

            lax.fori_loop(0, R, zero_row, 0)
            for j in range(R // L):
                cnt[pl.ds(L * j, L)] = zeros16

            c0 = jnp.maximum(rank(base) - 1, 0)
            c1 = rank(base + R)
            cs = lax.bitwise_and(c0, -2)

            @pl.when(cs < c1)
            def _():
                stage_idx(cs, 0)
                stage_dst(cs, 0)

            @pl.when(cs + 1 < c1)
            def _():
                stage_idx(cs + 1, 1)
                stage_dst(cs + 1, 1)

            @pl.when(cs < c1)
            def _():
                wait_stage(cs, 0)
                fire_gather(0)

            npairs = lax.div(c1 - cs + 1, 2)

            def pair_body(p, _):
                for b in range(2):
                    ci = cs + 2 * p + b

                    @pl.when(ci < c1)
                    def _():
                        wait_gather(b)

                        @pl.when(ci + 1 < c1)
                        def _():
                            wait_stage(ci + 1, 1 - b)
                            fire_gather(1 - b)

                        @pl.when(ci + 2 < c1)
                        def _():
                            stage_idx(ci + 2, b)

                        accumulate(b, base)

                        @pl.when(ci + 2 < c1)
                        def _():
                            stage_dst(ci + 2, b)

                return 0

            lax.fori_loop(0, npairs, pair_body, 0)

            def mean_body(m, _):
                cv = cnt[pl.ds(L * m, L)]
                iv = 1.0 / jnp.maximum(cv, 1.0)
                for l in range(L):
                    s = iv[l]
                    r = L * m + l
                    for j in range(DCH):
                        acc[r, pl.ds(L * j, L)] = acc[r, pl.ds(L * j, L)] * s
                return 0

            lax.fori_loop(0, R // L, mean_body, 0)
            pltpu.sync_copy(acc, out.at[pl.ds(base, R)])
            return 0

        lax.fori_loop(0, ranges_per_tile, range_body, 0)

    return seg


def _make_sage_matmul(M, relu, MB=1024):
    def body(x_ref, n_ref, ws_ref, wn_ref, b_ref, o_ref):
        acc = jnp.dot(x_ref[...], ws_ref[...], preferred_element_type=jnp.float32)
        acc = acc + jnp.dot(n_ref[...], wn_ref[...], preferred_element_type=jnp.float32)
        acc = acc + b_ref[...]
        if relu:
            acc = jnp.maximum(acc, 0.0)
        o_ref[...] = acc

    return pl.pallas_call(
        body,
        grid=(M // MB,),
        in_specs=[
            pl.BlockSpec((MB, D), lambda i: (i, 0)),
            pl.BlockSpec((MB, D), lambda i: (i, 0)),
            pl.BlockSpec((D, D), lambda i: (0, 0)),
            pl.BlockSpec((D, D), lambda i: (0, 0)),
            pl.BlockSpec((1, D), lambda i: (0, 0)),
        ],
        out_specs=pl.BlockSpec((MB, D), lambda i: (i, 0)),
        out_shape=jax.ShapeDtypeStruct((M, D), jnp.float32),
    )


def kernel(inputs, src1, dst1, src2, dst2, W_self1, W_neigh1, b1, W_self2, W_neigh2, b2):
    src1 = src1.astype(jnp.int32)
    dst1 = dst1.astype(jnp.int32)
    src2 = src2.astype(jnp.int32)
    dst2 = dst2.astype(jnp.int32)

    R = 128
    B = 128
    seg1 = _make_segmean(src1.shape[0], N1, R, B)
    seg2 = _make_segmean(src2.shape[0], N2, R, B)

    hn1 = seg1(inputs, src1, dst1, dst1[::B])
    h1 = _make_sage_matmul(N1, relu=True)(
        inputs[:N1], hn1, W_self1, W_neigh1, b1.reshape(1, D)
    )

    hn2 = seg2(h1, src2, dst2, dst2[::B])
    out = _make_sage_matmul(N2, relu=False)(
        h1[:N2], hn2, W_self2, W_neigh2, b2.reshape(1, D)
    )
    return out
